# Initial kernel scaffold; baseline (speedup 1.0000x reference)
import jax
import jax.numpy as jnp
from jax.experimental import pallas as pl


def kernel(n_id, memory, W1, b1, gamma, beta, W2, b2):
    raise NotImplementedError("write your pallas kernel here")


# speedup vs baseline: 1.4613x; 1.4613x over previous
import functools

import jax
import jax.numpy as jnp
from jax import lax
from jax.experimental import pallas as pl
from jax.experimental.pallas import tpu as pltpu
from jax.experimental.pallas import tpu_sc as plsc

NUM_NODES = 100000
D = 128
HIDDEN = 64
OUT = 10
B = 16384


_INFO = plsc.get_sparse_core_info()
_NC = _INFO.num_cores
_NS = _INFO.num_subcores
_NW = _NC * _NS
_BPW = B // _NW
_CHUNK = 128
_NCHUNK = _BPW // _CHUNK

_sc_mesh = plsc.VectorSubcoreMesh(core_axis_name="c", subcore_axis_name="s")


@functools.partial(
    pl.kernel,
    mesh=_sc_mesh,
    out_type=jax.ShapeDtypeStruct((B, D), jnp.float32),
    scratch_types=[
        pltpu.VMEM((_NCHUNK, _CHUNK), jnp.int32),
        pltpu.VMEM((_BPW, D), jnp.float32),
        pltpu.SemaphoreType.DMA,
    ],
)
def _sc_gather(table_hbm, idx_hbm, out_hbm, idx_v, rows_v, sem):
    wid = lax.axis_index("s") * _NC + lax.axis_index("c")
    base = wid * _BPW
    pltpu.sync_copy(idx_hbm.at[wid], idx_v)
    copies = []
    for j in range(_NCHUNK):
        copies.append(
            pltpu.async_copy(
                table_hbm.at[idx_v.at[j]],
                rows_v.at[pl.ds(j * _CHUNK, _CHUNK)],
                sem,
            )
        )
    for c in copies:
        c.wait()
    pltpu.sync_copy(rows_v, out_hbm.at[pl.ds(base, _BPW)])


_TILE = 2048
_T = B // _TILE


def _mlp_body(h_ref, w1_ref, b1_ref, gamma_ref, beta_ref, w2_ref, b2_ref,
              out_ref, x_scr, stat_scr):
    i = pl.program_id(0)

    @pl.when(i == 0)
    def _init():
        stat_scr[...] = jnp.zeros_like(stat_scr)

    @pl.when(i < _T)
    def _phase_a():
        x = jnp.dot(h_ref[...], w1_ref[...],
                    preferred_element_type=jnp.float32) + b1_ref[...]
        x = jnp.maximum(x, 0.0)
        x_scr[pl.ds(i * _TILE, _TILE), :] = x
        stat_scr[0:1, :] += jnp.sum(x, axis=0, keepdims=True)
        stat_scr[1:2, :] += jnp.sum(x * x, axis=0, keepdims=True)

    @pl.when(i >= _T)
    def _phase_b():
        j = i - _T
        mean = stat_scr[0:1, :] * (1.0 / B)
        var = stat_scr[1:2, :] * (1.0 / B) - mean * mean
        scale = gamma_ref[...] * lax.rsqrt(var + 1e-5)
        shift = beta_ref[...] - mean * scale
        x = x_scr[pl.ds(j * _TILE, _TILE), :]
        xn = x * scale + shift
        out_ref[...] = jnp.dot(xn, w2_ref[...],
                               preferred_element_type=jnp.float32) + b2_ref[...]


_mlp = pl.pallas_call(
    _mlp_body,
    grid=(2 * _T,),
    in_specs=[
        pl.BlockSpec((_TILE, D), lambda i: (jnp.minimum(i, _T - 1), 0)),
        pl.BlockSpec((D, HIDDEN), lambda i: (0, 0)),
        pl.BlockSpec((1, HIDDEN), lambda i: (0, 0)),
        pl.BlockSpec((1, HIDDEN), lambda i: (0, 0)),
        pl.BlockSpec((1, HIDDEN), lambda i: (0, 0)),
        pl.BlockSpec((HIDDEN, OUT), lambda i: (0, 0)),
        pl.BlockSpec((1, OUT), lambda i: (0, 0)),
    ],
    out_specs=pl.BlockSpec((_TILE, OUT), lambda i: (jnp.maximum(i - _T, 0), 0)),
    out_shape=jax.ShapeDtypeStruct((B, OUT), jnp.float32),
    scratch_shapes=[
        pltpu.VMEM((B, HIDDEN), jnp.float32),
        pltpu.VMEM((2, HIDDEN), jnp.float32),
    ],
    compiler_params=pltpu.CompilerParams(
        dimension_semantics=("arbitrary",),
    ),
)


def kernel(n_id, memory, W1, b1, gamma, beta, W2, b2):
    idx = n_id.astype(jnp.int32).reshape(_NW, _NCHUNK, _CHUNK)
    h = _sc_gather(memory, idx)
    return _mlp(h, W1, b1.reshape(1, HIDDEN), gamma.reshape(1, HIDDEN),
                beta.reshape(1, HIDDEN), W2, b2.reshape(1, OUT))


# speedup vs baseline: 1.8979x; 1.2988x over previous
import functools

import jax
import jax.numpy as jnp
from jax import lax
from jax.experimental import pallas as pl
from jax.experimental.pallas import tpu as pltpu
from jax.experimental.pallas import tpu_sc as plsc

NUM_NODES = 100000
D = 128
HIDDEN = 64
OUT = 10
B = 16384


_INFO = plsc.get_sparse_core_info()
_NC = _INFO.num_cores
_NS = _INFO.num_subcores
_NW = _NC * _NS
_BPW = B // _NW
_CHUNK = 128
_NCHUNK = _BPW // _CHUNK

_sc_mesh = plsc.VectorSubcoreMesh(core_axis_name="c", subcore_axis_name="s")


@functools.partial(
    pl.kernel,
    mesh=_sc_mesh,
    out_type=jax.ShapeDtypeStruct((B, D), jnp.float32),
    scratch_types=[
        pltpu.VMEM((_NCHUNK, _CHUNK), jnp.int32),
        pltpu.VMEM((_BPW, D), jnp.float32),
        pltpu.SemaphoreType.DMA,
        pltpu.SemaphoreType.DMA,
    ],
)
def _sc_gather(table_hbm, idx_hbm, out_hbm, idx_v, rows_v, gsem, wsem):
    wid = lax.axis_index("s") * _NC + lax.axis_index("c")
    base = wid * _BPW
    pltpu.sync_copy(idx_hbm.at[wid], idx_v)
    gathers = [
        pltpu.async_copy(
            table_hbm.at[idx_v.at[j]],
            rows_v.at[pl.ds(j * _CHUNK, _CHUNK)],
            gsem,
        )
        for j in range(_NCHUNK)
    ]
    writes = []
    for j in range(_NCHUNK):
        gathers[j].wait()
        writes.append(
            pltpu.async_copy(
                rows_v.at[pl.ds(j * _CHUNK, _CHUNK)],
                out_hbm.at[pl.ds(base + j * _CHUNK, _CHUNK)],
                wsem,
            )
        )
    for w in writes:
        w.wait()


_TILE = 4096
_T = B // _TILE


def _mlp_body(h_ref, w1_ref, b1_ref, gamma_ref, beta_ref, w2_ref, b2_ref,
              out_ref, x_scr, stat_scr):
    i = pl.program_id(0)

    @pl.when(i == 0)
    def _init():
        stat_scr[...] = jnp.zeros_like(stat_scr)

    @pl.when(i < _T)
    def _phase_a():
        x = jnp.dot(h_ref[...], w1_ref[...],
                    preferred_element_type=jnp.float32) + b1_ref[...]
        x = jnp.maximum(x, 0.0)
        x_scr[pl.ds(i * _TILE, _TILE), :] = x
        ones = jnp.ones((8, _TILE), jnp.float32)
        s = jnp.dot(ones, x, preferred_element_type=jnp.float32)
        q = jnp.dot(ones, x * x, preferred_element_type=jnp.float32)
        stat_scr[0:1, :] += s[0:1, :]
        stat_scr[1:2, :] += q[0:1, :]

    @pl.when(i >= _T)
    def _phase_b():
        j = i - _T
        mean = stat_scr[0:1, :] * (1.0 / B)
        var = stat_scr[1:2, :] * (1.0 / B) - mean * mean
        scale = gamma_ref[...] * lax.rsqrt(var + 1e-5)
        shift = beta_ref[...] - mean * scale
        w2p = w2_ref[...] * scale.reshape(HIDDEN, 1)
        c = jnp.dot(shift, w2_ref[...],
                    preferred_element_type=jnp.float32) + b2_ref[...]
        x = x_scr[pl.ds(j * _TILE, _TILE), :]
        out_t = lax.dot_general(w2p, x, (((0,), (1,)), ((), ())),
                                preferred_element_type=jnp.float32)
        out_ref[...] = out_t + c.reshape(OUT, 1)


_mlp = pl.pallas_call(
    _mlp_body,
    grid=(2 * _T,),
    in_specs=[
        pl.BlockSpec((_TILE, D), lambda i: (jnp.minimum(i, _T - 1), 0)),
        pl.BlockSpec((D, HIDDEN), lambda i: (0, 0)),
        pl.BlockSpec((1, HIDDEN), lambda i: (0, 0)),
        pl.BlockSpec((1, HIDDEN), lambda i: (0, 0)),
        pl.BlockSpec((1, HIDDEN), lambda i: (0, 0)),
        pl.BlockSpec((HIDDEN, OUT), lambda i: (0, 0)),
        pl.BlockSpec((1, OUT), lambda i: (0, 0)),
    ],
    out_specs=pl.BlockSpec((OUT, _TILE), lambda i: (0, jnp.maximum(i - _T, 0))),
    out_shape=jax.ShapeDtypeStruct((OUT, B), jnp.float32),
    scratch_shapes=[
        pltpu.VMEM((B, HIDDEN), jnp.float32),
        pltpu.VMEM((2, HIDDEN), jnp.float32),
    ],
    compiler_params=pltpu.CompilerParams(
        dimension_semantics=("arbitrary",),
    ),
)


def kernel(n_id, memory, W1, b1, gamma, beta, W2, b2):
    idx = n_id.astype(jnp.int32).reshape(_NW, _NCHUNK, _CHUNK)
    h = _sc_gather(memory, idx)
    out_t = _mlp(h, W1, b1.reshape(1, HIDDEN), gamma.reshape(1, HIDDEN),
                 beta.reshape(1, HIDDEN), W2, b2.reshape(1, OUT))
    return out_t.T
